# baseline (speedup 1.0000x reference)
import functools

import numpy as np

import jax
import jax.numpy as jnp
from jax import lax
from jax.experimental import pallas as pl
from jax.experimental.pallas import tpu as pltpu
from jax.experimental.pallas import tpu_sc as plsc

N_NODES = 10000
N_GROUND = 320000
D = 128
K = 3

NC = 2
NS = 16
NW = NC * NS

B_PER_W = N_GROUND // NW
CH = 40
N_CHUNK = B_PER_W // CH

_SIGN = np.uint32(0x80000000)


def _scale_body(w_ref, x_ref, o_ref):
    i = pl.program_id(0)
    o_ref[...] = x_ref[...] * (w_ref[i] * 2.0)


def _scale_table(x, weights):
    return pl.pallas_call(
        _scale_body,
        grid=(K,),
        in_specs=[
            pl.BlockSpec(memory_space=pltpu.SMEM),
            pl.BlockSpec((1, N_NODES, D), lambda i: (0, 0, 0)),
        ],
        out_specs=pl.BlockSpec((1, N_NODES, D), lambda i: (i, 0, 0)),
        out_shape=jax.ShapeDtypeStruct((K, N_NODES, D), jnp.float32),
    )(weights, x.reshape(1, N_NODES, D))


def _sc_body(t_hbm, idx_hbm, out_hbm,
             idx_v, r_a, r_b,
             ga0, ga1, ga2, gb0, gb1, gb2, osem_a, osem_b):
    wid = lax.axis_index("s") * NC + lax.axis_index("c")
    base = wid * B_PER_W

    r_bufs = (r_a, r_b)
    gsems = ((ga0, ga1, ga2), (gb0, gb1, gb2))
    osems = (osem_a, osem_b)

    for i in range(K):
        pltpu.sync_copy(idx_hbm.at[i, wid], idx_v.at[i])

    def gathers(c, b):
        return [
            pltpu.make_async_copy(
                t_hbm.at[i].at[idx_v.at[i, c]],
                r_bufs[b].at[pl.ds(i * CH, CH)],
                gsems[b][i],
            )
            for i in range(K)
        ]

    def outcp(c, b):
        return pltpu.make_async_copy(
            r_bufs[b].at[pl.ds(0, CH)],
            out_hbm.at[pl.ds(base + c * CH, CH)], osems[b])

    def compute(b):
        rv = r_bufs[b]
        ov = r_bufs[b]

        def row_body(r, rc):
            for c8 in range(D // 16):
                sl = pl.ds(c8 * 16, 16)
                z2 = rv[r, sl] + rv[r + CH, sl] + rv[r + 2 * CH, sl]
                zb = lax.bitcast_convert_type(z2, jnp.uint32)
                e = jnp.exp(lax.bitcast_convert_type(zb | _SIGN, jnp.float32))
                y = (1.0 - e) / (1.0 + e)
                yb = lax.bitcast_convert_type(y, jnp.uint32) | (zb & _SIGN)
                ov[r, sl] = lax.bitcast_convert_type(yb, jnp.float32)
            return rc

        lax.fori_loop(0, CH, row_body, 0)

    for cp in gathers(0, 0):
        cp.start()

    def outer(o, carry):
        for b in range(2):
            c = o * 2 + b
            nb = (b + 1) % 2

            @pl.when(c + 1 < N_CHUNK)
            def _():
                @pl.when(c >= 1)
                def _():
                    outcp(c - 1, nb).wait()

                for cp in gathers(c + 1, nb):
                    cp.start()

            for cp in gathers(c, b):
                cp.wait()

            compute(b)
            outcp(c, b).start()
        return carry

    lax.fori_loop(0, N_CHUNK // 2, outer, 0)
    outcp(N_CHUNK - 2, 0).wait()
    outcp(N_CHUNK - 1, 1).wait()


@jax.jit
def kernel(x, gather_indices, weights):
    t = _scale_table(x, weights)
    mesh = plsc.VectorSubcoreMesh(core_axis_name="c", subcore_axis_name="s")
    f = functools.partial(
        pl.kernel,
        mesh=mesh,
        out_type=jax.ShapeDtypeStruct((N_GROUND, D), jnp.float32),
        scratch_types=[
            pltpu.VMEM((K, N_CHUNK, CH), jnp.int32),
            pltpu.VMEM((K * CH, D), jnp.float32),
            pltpu.VMEM((K * CH, D), jnp.float32),
            pltpu.SemaphoreType.DMA,
            pltpu.SemaphoreType.DMA,
            pltpu.SemaphoreType.DMA,
            pltpu.SemaphoreType.DMA,
            pltpu.SemaphoreType.DMA,
            pltpu.SemaphoreType.DMA,
            pltpu.SemaphoreType.DMA,
            pltpu.SemaphoreType.DMA,
        ],
    )(_sc_body)
    return f(t, gather_indices.reshape(K, NW, N_CHUNK, CH))


# speedup vs baseline: 1.0011x; 1.0011x over previous
import functools

import numpy as np

import jax
import jax.numpy as jnp
from jax import lax
from jax.experimental import pallas as pl
from jax.experimental.pallas import tpu as pltpu
from jax.experimental.pallas import tpu_sc as plsc

N_NODES = 10000
N_GROUND = 320000
D = 128
K = 3

NC = 2
NS = 16
NW = NC * NS

B_PER_W = N_GROUND // NW
CH = 40
N_CHUNK = B_PER_W // CH

_SIGN = np.uint32(0x80000000)


def _sc_body(x_hbm, idx_hbm, w_hbm, out_hbm,
             i_a, i_b, w_v, r_a, r_b, o_a, o_b,
             isem_a, isem_b, gsem_a, gsem_b, osem_a, osem_b):
    wid = lax.axis_index("s") * NC + lax.axis_index("c")
    base = wid * B_PER_W

    i_bufs = (i_a, i_b)
    r_bufs = (r_a, r_b)
    o_bufs = (o_a, o_b)
    isems = (isem_a, isem_b)
    gsems = (gsem_a, gsem_b)
    osems = (osem_a, osem_b)

    pltpu.sync_copy(w_hbm, w_v)
    w0 = w_v[0, :] + w_v[0, :]
    w1 = w_v[1, :] + w_v[1, :]
    w2 = w_v[2, :] + w_v[2, :]

    def idxcps(c, b):
        return [
            pltpu.make_async_copy(
                idx_hbm.at[i, wid, c, 0],
                i_bufs[b].at[pl.ds(i * CH, CH)],
                isems[b],
            )
            for i in range(K)
        ]

    def gather(c, b):
        return pltpu.make_async_copy(x_hbm.at[i_bufs[b]], r_bufs[b], gsems[b])

    def outcp(c, b):
        return pltpu.make_async_copy(
            o_bufs[b], out_hbm.at[pl.ds(base + c * CH, CH)], osems[b])

    def compute(b):
        rv = r_bufs[b]
        ov = o_bufs[b]

        def row_body(r, rc):
            for c8 in range(D // 16):
                sl = pl.ds(c8 * 16, 16)
                z2 = (rv[r, sl] * w0 + rv[r + CH, sl] * w1
                      + rv[r + 2 * CH, sl] * w2)
                zb = lax.bitcast_convert_type(z2, jnp.uint32)
                e = jnp.exp(lax.bitcast_convert_type(zb | _SIGN, jnp.float32))
                y = (1.0 - e) / (1.0 + e)
                yb = lax.bitcast_convert_type(y, jnp.uint32) | (zb & _SIGN)
                ov[r, sl] = lax.bitcast_convert_type(yb, jnp.float32)
            return rc

        lax.fori_loop(0, CH, row_body, 0)

    for cp in idxcps(0, 0):
        cp.start()
    for cp in idxcps(0, 0):
        cp.wait()
    gather(0, 0).start()
    for cp in idxcps(1, 1):
        cp.start()

    def outer(o, carry):
        for b in range(2):
            c = o * 2 + b
            nb = (b + 1) % 2

            @pl.when(c + 1 < N_CHUNK)
            def _():
                for cp in idxcps(c + 1, nb):
                    cp.wait()
                gather(c + 1, nb).start()

            gather(c, b).wait()

            @pl.when(c + 2 < N_CHUNK)
            def _():
                for cp in idxcps(c + 2, b):
                    cp.start()

            @pl.when(c >= 2)
            def _():
                outcp(c - 2, b).wait()

            compute(b)
            outcp(c, b).start()
        return carry

    lax.fori_loop(0, N_CHUNK // 2, outer, 0)
    outcp(N_CHUNK - 2, 0).wait()
    outcp(N_CHUNK - 1, 1).wait()


@jax.jit
def kernel(x, gather_indices, weights):
    mesh = plsc.VectorSubcoreMesh(core_axis_name="c", subcore_axis_name="s")
    f = functools.partial(
        pl.kernel,
        mesh=mesh,
        out_type=jax.ShapeDtypeStruct((N_GROUND, D), jnp.float32),
        scratch_types=[
            pltpu.VMEM((K * CH,), jnp.int32),
            pltpu.VMEM((K * CH,), jnp.int32),
            pltpu.VMEM((K, 16), jnp.float32),
            pltpu.VMEM((K * CH, D), jnp.float32),
            pltpu.VMEM((K * CH, D), jnp.float32),
            pltpu.VMEM((CH, D), jnp.float32),
            pltpu.VMEM((CH, D), jnp.float32),
            pltpu.SemaphoreType.DMA,
            pltpu.SemaphoreType.DMA,
            pltpu.SemaphoreType.DMA,
            pltpu.SemaphoreType.DMA,
            pltpu.SemaphoreType.DMA,
            pltpu.SemaphoreType.DMA,
        ],
    )(_sc_body)
    return f(
        x,
        gather_indices.reshape(K, NW, N_CHUNK, 1, CH),
        jnp.broadcast_to(weights.reshape(K, 1), (K, 16)),
    )


# speedup vs baseline: 1.0371x; 1.0360x over previous
import functools

import numpy as np

import jax
import jax.numpy as jnp
from jax import lax
from jax.experimental import pallas as pl
from jax.experimental.pallas import tpu as pltpu
from jax.experimental.pallas import tpu_sc as plsc

N_NODES = 10000
N_GROUND = 320000
D = 128
K = 3

NC = 2
NS = 16
NW = NC * NS

B_PER_W = N_GROUND // NW
CH = 40
N_CHUNK = B_PER_W // CH

_SIGN = np.uint32(0x80000000)


def _sc_body(x_hbm, idx_hbm, w_hbm, out_hbm,
             idx_v, w_v, r_a, r_b, o_a, o_b,
             gsem_a, gsem_b, osem_a, osem_b):
    wid = lax.axis_index("s") * NC + lax.axis_index("c")
    base = wid * B_PER_W

    r_bufs = (r_a, r_b)
    o_bufs = (o_a, o_b)
    gsems = (gsem_a, gsem_b)
    osems = (osem_a, osem_b)

    pltpu.sync_copy(idx_hbm.at[wid], idx_v)
    pltpu.sync_copy(w_hbm, w_v)
    w0 = w_v[0, :] + w_v[0, :]
    w1 = w_v[1, :] + w_v[1, :]
    w2 = w_v[2, :] + w_v[2, :]

    def gather(c, b):
        return pltpu.make_async_copy(x_hbm.at[idx_v.at[c]], r_bufs[b], gsems[b])

    def outcp(c, b):
        return pltpu.make_async_copy(
            o_bufs[b], out_hbm.at[pl.ds(base + c * CH, CH)], osems[b])

    def compute(b):
        rv = r_bufs[b]
        ov = o_bufs[b]

        def row_body(r, rc):
            for c8 in range(D // 16):
                sl = pl.ds(c8 * 16, 16)
                z2 = (rv[r, sl] * w0 + rv[r + CH, sl] * w1
                      + rv[r + 2 * CH, sl] * w2)
                zb = lax.bitcast_convert_type(z2, jnp.uint32)
                e = jnp.exp(lax.bitcast_convert_type(zb | _SIGN, jnp.float32))
                y = (1.0 - e) / (1.0 + e)
                yb = lax.bitcast_convert_type(y, jnp.uint32) | (zb & _SIGN)
                ov[r, sl] = lax.bitcast_convert_type(yb, jnp.float32)
            return rc

        lax.fori_loop(0, CH, row_body, 0)

    gather(0, 0).start()

    def outer(o, carry):
        for b in range(2):
            c = o * 2 + b
            nb = (b + 1) % 2

            @pl.when(c + 1 < N_CHUNK)
            def _():
                gather(c + 1, nb).start()

            gather(c, b).wait()

            @pl.when(c >= 2)
            def _():
                outcp(c - 2, b).wait()

            compute(b)
            outcp(c, b).start()
        return carry

    lax.fori_loop(0, N_CHUNK // 2, outer, 0)
    outcp(N_CHUNK - 2, 0).wait()
    outcp(N_CHUNK - 1, 1).wait()


@jax.jit
def kernel(x, gather_indices, weights):
    idx_r = (gather_indices.reshape(K, NW, N_CHUNK, CH)
             .transpose(1, 2, 0, 3)
             .reshape(NW, N_CHUNK, K * CH))
    w_b = jnp.broadcast_to(weights.reshape(K, 1), (K, 16))
    mesh = plsc.VectorSubcoreMesh(core_axis_name="c", subcore_axis_name="s")
    f = functools.partial(
        pl.kernel,
        mesh=mesh,
        out_type=jax.ShapeDtypeStruct((N_GROUND, D), jnp.float32),
        scratch_types=[
            pltpu.VMEM((N_CHUNK, K * CH), jnp.int32),
            pltpu.VMEM((K, 16), jnp.float32),
            pltpu.VMEM((K * CH, D), jnp.float32),
            pltpu.VMEM((K * CH, D), jnp.float32),
            pltpu.VMEM((CH, D), jnp.float32),
            pltpu.VMEM((CH, D), jnp.float32),
            pltpu.SemaphoreType.DMA,
            pltpu.SemaphoreType.DMA,
            pltpu.SemaphoreType.DMA,
            pltpu.SemaphoreType.DMA,
        ],
    )(_sc_body)
    return f(x, idx_r, w_b)


# speedup vs baseline: 1.0799x; 1.0413x over previous
import functools

import numpy as np

import jax
import jax.numpy as jnp
from jax import lax
from jax.experimental import pallas as pl
from jax.experimental.pallas import tpu as pltpu
from jax.experimental.pallas import tpu_sc as plsc

N_NODES = 10000
N_GROUND = 320000
D = 128
K = 3

NC = 2
NS = 16
NW = NC * NS

B_PER_W = N_GROUND // NW
CH = 40
N_CHUNK = B_PER_W // CH

_SIGN = np.uint32(0x80000000)


def _scale_body(w_ref, x_ref, o_ref):
    i = pl.program_id(0)
    o_ref[...] = x_ref[...] * (w_ref[i] * 2.0)


def _scale_table(x, weights):
    return pl.pallas_call(
        _scale_body,
        grid=(K,),
        in_specs=[
            pl.BlockSpec(memory_space=pltpu.SMEM),
            pl.BlockSpec((N_NODES, D), lambda i: (0, 0)),
        ],
        out_specs=pl.BlockSpec((N_NODES, D), lambda i: (i, 0)),
        out_shape=jax.ShapeDtypeStruct((K * N_NODES, D), jnp.float32),
    )(weights, x)


def _sc_body(t_hbm, idx_hbm, out_hbm,
             idx_v, r_a, r_b, o_a, o_b,
             gsem_a, gsem_b, osem_a, osem_b):
    wid = lax.axis_index("s") * NC + lax.axis_index("c")
    base = wid * B_PER_W

    r_bufs = (r_a, r_b)
    o_bufs = (o_a, o_b)
    gsems = (gsem_a, gsem_b)
    osems = (osem_a, osem_b)

    pltpu.sync_copy(idx_hbm.at[wid], idx_v)

    def gather(c, b):
        return pltpu.make_async_copy(t_hbm.at[idx_v.at[c]], r_bufs[b], gsems[b])

    def outcp(c, b):
        return pltpu.make_async_copy(
            o_bufs[b], out_hbm.at[pl.ds(base + c * CH, CH)], osems[b])

    def compute(b):
        rv = r_bufs[b]
        ov = o_bufs[b]

        def row_body(r, rc):
            for c8 in range(D // 16):
                sl = pl.ds(c8 * 16, 16)
                z2 = rv[r, sl] + rv[r + CH, sl] + rv[r + 2 * CH, sl]
                zb = lax.bitcast_convert_type(z2, jnp.uint32)
                e = jnp.exp(lax.bitcast_convert_type(zb | _SIGN, jnp.float32))
                y = (1.0 - e) / (1.0 + e)
                yb = lax.bitcast_convert_type(y, jnp.uint32) | (zb & _SIGN)
                ov[r, sl] = lax.bitcast_convert_type(yb, jnp.float32)
            return rc

        lax.fori_loop(0, CH, row_body, 0)

    gather(0, 0).start()

    def outer(o, carry):
        for b in range(2):
            c = o * 2 + b
            nb = (b + 1) % 2

            @pl.when(c + 1 < N_CHUNK)
            def _():
                gather(c + 1, nb).start()

            gather(c, b).wait()

            @pl.when(c >= 2)
            def _():
                outcp(c - 2, b).wait()

            compute(b)
            outcp(c, b).start()
        return carry

    lax.fori_loop(0, N_CHUNK // 2, outer, 0)
    outcp(N_CHUNK - 2, 0).wait()
    outcp(N_CHUNK - 1, 1).wait()


@jax.jit
def kernel(x, gather_indices, weights):
    idx_s = gather_indices + (jnp.arange(K, dtype=jnp.int32) * N_NODES)[:, None]
    idx_r = (idx_s.reshape(K, NW, N_CHUNK, CH)
             .transpose(1, 2, 0, 3)
             .reshape(NW, N_CHUNK, K * CH))
    t = _scale_table(x, weights)
    mesh = plsc.VectorSubcoreMesh(core_axis_name="c", subcore_axis_name="s")
    f = functools.partial(
        pl.kernel,
        mesh=mesh,
        out_type=jax.ShapeDtypeStruct((N_GROUND, D), jnp.float32),
        scratch_types=[
            pltpu.VMEM((N_CHUNK, K * CH), jnp.int32),
            pltpu.VMEM((K * CH, D), jnp.float32),
            pltpu.VMEM((K * CH, D), jnp.float32),
            pltpu.VMEM((CH, D), jnp.float32),
            pltpu.VMEM((CH, D), jnp.float32),
            pltpu.SemaphoreType.DMA,
            pltpu.SemaphoreType.DMA,
            pltpu.SemaphoreType.DMA,
            pltpu.SemaphoreType.DMA,
        ],
    )(_sc_body)
    return f(t, idx_r)
